# baseline (speedup 1.0000x reference)
import dataclasses
import functools

import jax
import jax.numpy as jnp
from jax import lax
from jax.experimental import pallas as pl
from jax.experimental.pallas import tpu as pltpu
from jax.experimental.pallas import tpu_sc as plsc

N = 50000
E = 800000
DIN = 12
H = 128
DOUT = 2

N_PAD = 51200
E_PAD = 819200
EPW = 25600
SCAN = 3200
NCHUNK = EPW // SCAN
NVEC = SCAN // 16
NRC = 8
RANGE = 6400
CAPB = 4096
CAP_EFF = CAPB - 128
BW_FLAT = NRC * CAPB
BCH = 1024
BLK = 128
ROWS_PT = RANGE // 16

ROW_BLK = 2048

_mesh = plsc.VectorSubcoreMesh(core_axis_name="c", subcore_axis_name="s",
                               num_cores=2, num_subcores=16)
_cp = pltpu.CompilerParams()
if "needs_layout_passes" in pltpu.CompilerParams.__dataclass_fields__:
    _cp = dataclasses.replace(_cp, needs_layout_passes=False)


def _dense0_body(x_ref, W_ref, o_ref):
    o_ref[...] = jnp.dot(x_ref[...], W_ref[...],
                         preferred_element_type=jnp.float32,
                         precision=lax.Precision.HIGHEST)


def _dense2_body(a_ref, b_ref, W_ref, o_ref):
    h = jnp.maximum(a_ref[...] + b_ref[...], 0.0)
    o_ref[...] = jnp.dot(h, W_ref[...], preferred_element_type=jnp.float32,
                         precision=lax.Precision.HIGHEST)


def _head_body(a3_ref, b3_ref, A1_ref, ab1_ref, A2_ref, ab2_ref, A3_ref,
               ab3_ref, A4_ref, ab4_ref, o_ref):
    h = jnp.maximum(a3_ref[...] + b3_ref[...], 0.0)
    for W_ref, b_ref in ((A1_ref, ab1_ref), (A2_ref, ab2_ref),
                         (A3_ref, ab3_ref)):
        h = jnp.maximum(
            jnp.dot(h, W_ref[...], preferred_element_type=jnp.float32,
                    precision=lax.Precision.HIGHEST) + b_ref[...], 0.0)
    o_ref[...] = (jnp.dot(h, A4_ref[...], preferred_element_type=jnp.float32,
                          precision=lax.Precision.HIGHEST) + ab4_ref[...])


def _row_blocked(body, out_dim, x, *full_args):
    grid = (N_PAD // ROW_BLK,)
    in_specs = [pl.BlockSpec((ROW_BLK, x.shape[1]), lambda i: (i, 0))]
    for a in full_args:
        in_specs.append(
            pl.BlockSpec(a.shape, lambda i, _r=len(a.shape): (0,) * _r))
    return pl.pallas_call(
        body,
        grid=grid,
        in_specs=in_specs,
        out_specs=pl.BlockSpec((ROW_BLK, out_dim), lambda i: (i, 0)),
        out_shape=jax.ShapeDtypeStruct((N_PAD, out_dim), jnp.float32),
    )(x, *full_args)


def _sort_body(dst_hbm, src_hbm, w_hbm, bsrc_hbm, bdrel_hbm, bw_hbm, cnt_hbm,
               dstbuf, srcbuf, wbuf, sg_src, sg_drel, sg_w, cntbuf):
    c = lax.axis_index("c")
    s = lax.axis_index("s")
    wid = c * 16 + s
    ebase = wid * EPW
    iota = lax.iota(jnp.int32, 16)
    zi = jnp.zeros((16,), jnp.int32)
    zf = jnp.zeros((16,), jnp.float32)

    def chunk_body(ci, ptrs):
        off = ebase + ci * SCAN
        pltpu.sync_copy(dst_hbm.at[pl.ds(off, SCAN)], dstbuf)
        pltpu.sync_copy(src_hbm.at[pl.ds(off, SCAN)], srcbuf)
        pltpu.sync_copy(w_hbm.at[pl.ds(off, SCAN)], wbuf)
        new_ptrs = []
        for b in range(NRC):
            def vec_body(j, ptr, _b=b):
                bb = j * 16
                d = dstbuf[pl.ds(bb, 16)]
                sv = srcbuf[pl.ds(bb, 16)]
                wv = wbuf[pl.ds(bb, 16)]
                drel = d - _b * RANGE
                m = (drel >= 0) & (drel < RANGE)
                mi = jnp.where(m, 1, 0).astype(jnp.int32)
                inc = plsc.cumsum(mi)
                pos = (_b * CAPB + ptr) + inc - 1
                plsc.store_scatter(sg_src, [pos], sv, mask=m)
                plsc.store_scatter(sg_drel, [pos], drel, mask=m)
                plsc.store_scatter(sg_w, [pos], wv, mask=m)
                return jnp.minimum(ptr + jnp.sum(mi), CAP_EFF)

            new_ptrs.append(lax.fori_loop(0, NVEC, vec_body, ptrs[b]))
        return tuple(new_ptrs)

    ptrs = lax.fori_loop(0, NCHUNK, chunk_body,
                         tuple(jnp.int32(0) for _ in range(NRC)))

    for b in range(NRC):
        p = ptrs[b]
        p0 = (p // 16) * 16
        for q in range(8):
            idx16 = iota + (b * CAPB + p0 + q * 16)
            mq = (iota + p0 + q * 16) >= p
            plsc.store_scatter(sg_src, [idx16], zi, mask=mq)
            plsc.store_scatter(sg_drel, [idx16], zi, mask=mq)
            plsc.store_scatter(sg_w, [idx16], zf, mask=mq)
        plsc.store_scatter(cntbuf, [jnp.full((16,), b, jnp.int32)],
                           jnp.broadcast_to(p, (16,)).astype(jnp.int32),
                           mask=(iota == b))
    pltpu.sync_copy(sg_src, bsrc_hbm.at[wid])
    pltpu.sync_copy(sg_drel, bdrel_hbm.at[wid])
    pltpu.sync_copy(sg_w, bw_hbm.at[wid])
    pltpu.sync_copy(cntbuf, cnt_hbm.at[wid])


_sort = pl.kernel(
    _sort_body,
    out_type=[jax.ShapeDtypeStruct((32, BW_FLAT), jnp.int32),
              jax.ShapeDtypeStruct((32, BW_FLAT), jnp.int32),
              jax.ShapeDtypeStruct((32, BW_FLAT), jnp.float32),
              jax.ShapeDtypeStruct((32, 16), jnp.int32)],
    mesh=_mesh,
    compiler_params=_cp,
    scratch_types=[
        pltpu.VMEM((SCAN,), jnp.int32),
        pltpu.VMEM((SCAN,), jnp.int32),
        pltpu.VMEM((SCAN,), jnp.float32),
        pltpu.VMEM((BW_FLAT,), jnp.int32),
        pltpu.VMEM((BW_FLAT,), jnp.int32),
        pltpu.VMEM((BW_FLAT,), jnp.float32),
        pltpu.VMEM((16,), jnp.int32),
    ],
)


def _spmm_body(S_hbm, bsrc_hbm, bdrel_hbm, bw_hbm, cnt_hbm, z_hbm, out_hbm,
               esrc, edrel, ew, fsrc, fidx, rowbuf, cbuf, acc, gsem, ssem):
    c = lax.axis_index("c")
    s = lax.axis_index("s")
    iota = lax.iota(jnp.int32, 16)

    def stage(koff, slot):
        for q2 in range(BLK // 16):
            fsrc[slot, pl.ds(q2 * 16, 16)] = esrc[pl.ds(koff + q2 * 16, 16)]
            fidx[slot, pl.ds(q2 * 16, 16)] = edrel[pl.ds(koff + q2 * 16, 16)]

    def gather_start(slot):
        pltpu.async_copy(S_hbm.at[fsrc.at[slot]],
                         rowbuf.at[pl.ds(slot * BLK, BLK)], gsem)

    def gather_wait(slot):
        pltpu.make_async_copy(S_hbm.at[fsrc.at[slot]],
                              rowbuf.at[pl.ds(slot * BLK, BLK)], gsem).wait()

    def scatter_start(slot):
        pltpu.async_copy(rowbuf.at[pl.ds(slot * BLK, BLK)],
                         acc.at[fidx.at[slot]], ssem, add=True)

    def scatter_wait(slot):
        pltpu.make_async_copy(rowbuf.at[pl.ds(slot * BLK, BLK)],
                              acc.at[fidx.at[slot]], ssem).wait()

    def scale(slot, woff):
        def row_body(i, _3):
            wrow = plsc.load_gather(ew, [jnp.full((16,), woff + i, jnp.int32)])
            r = slot * BLK + i
            for q3 in range(8):
                rowbuf[r, pl.ds(q3 * 16, 16)] = (
                    rowbuf[r, pl.ds(q3 * 16, 16)] * wrow)
            return 0
        lax.fori_loop(0, BLK, row_body, 0, unroll=4)

    for p in range(4):
        rc = c * 4 + p
        lo = rc * RANGE
        b0 = rc * CAPB
        pltpu.sync_copy(z_hbm.at[pl.ds(s * ROWS_PT, ROWS_PT)],
                        acc.at[pl.ds(s * ROWS_PT, ROWS_PT)])
        plsc.subcore_barrier()
        w0 = 2 * s
        pltpu.sync_copy(cnt_hbm.at[w0], cbuf.at[0])
        pltpu.sync_copy(cnt_hbm.at[w0 + 1], cbuf.at[1])
        pltpu.sync_copy(bsrc_hbm.at[w0, pl.ds(b0, CAPB)],
                        esrc.at[pl.ds(0, CAPB)])
        pltpu.sync_copy(bsrc_hbm.at[w0 + 1, pl.ds(b0, CAPB)],
                        esrc.at[pl.ds(CAPB, CAPB)])
        pltpu.sync_copy(bdrel_hbm.at[w0, pl.ds(b0, CAPB)],
                        edrel.at[pl.ds(0, CAPB)])
        pltpu.sync_copy(bdrel_hbm.at[w0 + 1, pl.ds(b0, CAPB)],
                        edrel.at[pl.ds(CAPB, CAPB)])
        pltpu.sync_copy(bw_hbm.at[w0, pl.ds(b0, CAPB)], ew.at[pl.ds(0, CAPB)])
        pltpu.sync_copy(bw_hbm.at[w0 + 1, pl.ds(b0, CAPB)],
                        ew.at[pl.ds(CAPB, CAPB)])
        cnt0 = jnp.sum(jnp.where(iota == rc, cbuf[0, pl.ds(0, 16)], 0))
        cnt1 = jnp.sum(jnp.where(iota == rc, cbuf[1, pl.ds(0, 16)], 0))
        nb0 = (cnt0 + BLK - 1) // BLK
        nbt = nb0 + (cnt1 + BLK - 1) // BLK

        def off_of(k, _nb0=nb0):
            return jnp.where(k < _nb0, k * BLK, CAPB + (k - _nb0) * BLK)

        @pl.when(nbt > 0)
        def _():
            stage(off_of(0), 0)
            gather_start(0)

        def blk_body(k, _, _nbt=nbt, _off=off_of):
            par = k % 2
            nxt = 1 - par

            @pl.when(k >= 1)
            def _():
                scatter_wait(nxt)

            @pl.when(k + 1 < _nbt)
            def _():
                stage(_off(k + 1), nxt)
                gather_start(nxt)

            gather_wait(par)
            scale(par, _off(k))
            scatter_start(par)
            return 0

        lax.fori_loop(0, nbt, blk_body, 0)

        @pl.when(nbt > 0)
        def _():
            scatter_wait((nbt - 1) % 2)

        plsc.subcore_barrier()
        pltpu.sync_copy(acc.at[pl.ds(s * ROWS_PT, ROWS_PT)],
                        out_hbm.at[pl.ds(lo + s * ROWS_PT, ROWS_PT)])
        plsc.subcore_barrier()


_spmm = pl.kernel(
    _spmm_body,
    out_type=jax.ShapeDtypeStruct((N_PAD, H), jnp.float32),
    mesh=_mesh,
    compiler_params=_cp,
    scratch_types=[
        pltpu.VMEM((2 * CAPB,), jnp.int32),
        pltpu.VMEM((2 * CAPB,), jnp.int32),
        pltpu.VMEM((2 * CAPB,), jnp.float32),
        pltpu.VMEM((2, BLK), jnp.int32),
        pltpu.VMEM((2, BLK), jnp.int32),
        pltpu.VMEM((2 * BLK, H), jnp.float32),
        pltpu.VMEM((2, 16), jnp.int32),
        pltpu.VMEM_SHARED((RANGE, H), jnp.float32),
        pltpu.SemaphoreType.DMA,
        pltpu.SemaphoreType.DMA,
    ],
)


def kernel(x, edge_index, edge_weight, W1, b1, W2, b2, W3, b3,
           A1, ab1, A2, ab2, A3, ab3, A4, ab4):
    npad = E_PAD - E
    pad_dst = (jnp.arange(npad, dtype=jnp.int32) % NRC) * RANGE
    dst = jnp.concatenate([edge_index[0], pad_dst])
    src = jnp.concatenate([edge_index[1], jnp.zeros((npad,), jnp.int32)])
    w = jnp.concatenate([edge_weight, jnp.zeros((npad,), jnp.float32)])
    x_pad = jnp.zeros((N_PAD, 16), jnp.float32).at[:N, :DIN].set(x)
    z128 = jnp.zeros((RANGE, H), jnp.float32)
    W1p = jnp.zeros((16, H), jnp.float32).at[:DIN].set(W1)

    bsrc, bdrel, bw, cnts = _sort(dst, src, w)

    s1 = _row_blocked(_dense0_body, H, x_pad, W1p)
    a1 = _spmm(s1, bsrc, bdrel, bw, cnts, z128)
    s2 = _row_blocked(_dense2_body, H, a1, b1.reshape(1, H), W2)
    a2 = _spmm(s2, bsrc, bdrel, bw, cnts, z128)
    s3 = _row_blocked(_dense2_body, H, a2, b2.reshape(1, H), W3)
    a3 = _spmm(s3, bsrc, bdrel, bw, cnts, z128)
    scores = _row_blocked(_head_body, DOUT, a3,
                          b3.reshape(1, H), A1, ab1.reshape(1, H),
                          A2, ab2.reshape(1, H), A3, ab3.reshape(1, H),
                          A4, ab4.reshape(1, DOUT))
    return scores[:N]


# speedup vs baseline: 1.4332x; 1.4332x over previous
import dataclasses
import functools

import jax
import jax.numpy as jnp
from jax import lax
from jax.experimental import pallas as pl
from jax.experimental.pallas import tpu as pltpu
from jax.experimental.pallas import tpu_sc as plsc

N = 50000
E = 800000
DIN = 12
H = 128
DOUT = 2

N_PAD = 51200
E_PAD = 819200
EPT = 51200
SCAN = 3200
NCHUNK = EPT // SCAN
NVEC = SCAN // 16
BLK = 128

ROW_BLK = 2048


def _dense0_body(x_ref, W_ref, o_ref):
    o_ref[...] = jnp.dot(x_ref[...], W_ref[...],
                         preferred_element_type=jnp.float32,
                         precision=lax.Precision.HIGHEST)


def _dense2_body(a_ref, b_ref, W_ref, o_ref):
    h = jnp.maximum(a_ref[...] + b_ref[...], 0.0)
    o_ref[...] = jnp.dot(h, W_ref[...], preferred_element_type=jnp.float32,
                         precision=lax.Precision.HIGHEST)


def _head_body(a3_ref, b3_ref, A1_ref, ab1_ref, A2_ref, ab2_ref, A3_ref,
               ab3_ref, A4_ref, ab4_ref, o_ref):
    h = jnp.maximum(a3_ref[...] + b3_ref[...], 0.0)
    for W_ref, b_ref in ((A1_ref, ab1_ref), (A2_ref, ab2_ref),
                         (A3_ref, ab3_ref)):
        h = jnp.maximum(
            jnp.dot(h, W_ref[...], preferred_element_type=jnp.float32,
                    precision=lax.Precision.HIGHEST) + b_ref[...], 0.0)
    o_ref[...] = (jnp.dot(h, A4_ref[...], preferred_element_type=jnp.float32,
                          precision=lax.Precision.HIGHEST) + ab4_ref[...])


def _row_blocked(body, out_dim, x, *full_args):
    grid = (N_PAD // ROW_BLK,)
    in_specs = [pl.BlockSpec((ROW_BLK, x.shape[1]), lambda i: (i, 0))]
    for a in full_args:
        in_specs.append(
            pl.BlockSpec(a.shape, lambda i, _r=len(a.shape): (0,) * _r))
    return pl.pallas_call(
        body,
        grid=grid,
        in_specs=in_specs,
        out_specs=pl.BlockSpec((ROW_BLK, out_dim), lambda i: (i, 0)),
        out_shape=jax.ShapeDtypeStruct((N_PAD, out_dim), jnp.float32),
    )(x, *full_args)


def _make_spmm(D, RANGE, NPASS, CAP):
    ROWS_PT = RANGE // 16
    CALLOC = CAP + 144
    NQ = D // 16
    mesh = plsc.VectorSubcoreMesh(core_axis_name="c", subcore_axis_name="s",
                                  num_cores=2, num_subcores=16)

    def body(S_hbm, dst_hbm, src_hbm, w_hbm, z_hbm, out_hbm,
             dstbuf, srcbuf, wbuf, st_src, st_drel, st_w,
             fsrc, fidx, rowbuf, acc, gsem, ssem):
        c = lax.axis_index("c")
        s = lax.axis_index("s")
        ebase = s * EPT
        iota = lax.iota(jnp.int32, 16)
        zi = jnp.zeros((16,), jnp.int32)
        zf = jnp.zeros((16,), jnp.float32)
        for p in range(NPASS):
            lo = (c * NPASS + p) * RANGE
            pltpu.sync_copy(z_hbm.at[pl.ds(s * ROWS_PT, ROWS_PT)],
                            acc.at[pl.ds(s * ROWS_PT, ROWS_PT)])
            plsc.subcore_barrier()

            def vec_body(j, ptr, _lo=lo):
                b = j * 16
                d = dstbuf[pl.ds(b, 16)]
                sv = srcbuf[pl.ds(b, 16)]
                wv = wbuf[pl.ds(b, 16)]
                drel = d - _lo
                m = (drel >= 0) & (drel < RANGE)
                mi = jnp.where(m, 1, 0).astype(jnp.int32)
                inc = plsc.cumsum(mi)
                pos = ptr + inc - 1
                plsc.store_scatter(st_src, [pos], sv, mask=m)
                plsc.store_scatter(st_drel, [pos], drel, mask=m)
                plsc.store_scatter(st_w, [pos], wv, mask=m)
                cnt = jnp.sum(mi)
                return jnp.minimum(ptr + cnt, CAP)

            def chunk_body(ci, ptr):
                off = ebase + ci * SCAN
                pltpu.sync_copy(dst_hbm.at[pl.ds(off, SCAN)], dstbuf)
                pltpu.sync_copy(src_hbm.at[pl.ds(off, SCAN)], srcbuf)
                pltpu.sync_copy(w_hbm.at[pl.ds(off, SCAN)], wbuf)
                return lax.fori_loop(0, NVEC, vec_body, ptr)

            ptr = lax.fori_loop(0, NCHUNK, chunk_body, jnp.int32(0))

            p0 = (ptr // 16) * 16
            for q in range(8):
                idx16 = iota + p0 + q * 16
                mq = idx16 >= ptr
                plsc.store_scatter(st_src, [idx16], zi, mask=mq)
                plsc.store_scatter(st_drel, [idx16], zi, mask=mq)
                plsc.store_scatter(st_w, [idx16], zf, mask=mq)
            nblk = (ptr + BLK - 1) // BLK

            def stage(kb, slot):
                for q2 in range(BLK // 16):
                    fsrc[slot, pl.ds(q2 * 16, 16)] = (
                        st_src[pl.ds(kb + q2 * 16, 16)])
                    fidx[slot, pl.ds(q2 * 16, 16)] = (
                        st_drel[pl.ds(kb + q2 * 16, 16)])

            def gather_start(slot):
                pltpu.async_copy(S_hbm.at[fsrc.at[slot]],
                                 rowbuf.at[pl.ds(slot * BLK, BLK)], gsem)

            def gather_wait(slot):
                pltpu.make_async_copy(
                    S_hbm.at[fsrc.at[slot]],
                    rowbuf.at[pl.ds(slot * BLK, BLK)], gsem).wait()

            def scatter_start(slot):
                pltpu.async_copy(rowbuf.at[pl.ds(slot * BLK, BLK)],
                                 acc.at[fidx.at[slot]], ssem, add=True)

            def scatter_wait(slot):
                pltpu.make_async_copy(rowbuf.at[pl.ds(slot * BLK, BLK)],
                                      acc.at[fidx.at[slot]], ssem).wait()

            def scale(slot, woff):
                def row_body(i, _2):
                    wrow = plsc.load_gather(
                        st_w, [jnp.full((16,), woff + i, jnp.int32)])
                    r = slot * BLK + i
                    for q3 in range(NQ):
                        rowbuf[r, pl.ds(q3 * 16, 16)] = (
                            rowbuf[r, pl.ds(q3 * 16, 16)] * wrow)
                    return 0
                lax.fori_loop(0, BLK, row_body, 0, unroll=2)

            @pl.when(nblk > 0)
            def _():
                stage(0, 0)
                gather_start(0)

            def blk_body(k, _, _nblk=nblk):
                par = k % 2
                nxt = 1 - par

                @pl.when(k >= 1)
                def _():
                    scatter_wait(nxt)

                @pl.when(k + 1 < _nblk)
                def _():
                    stage((k + 1) * BLK, nxt)
                    gather_start(nxt)

                gather_wait(par)
                scale(par, k * BLK)
                scatter_start(par)
                return 0

            lax.fori_loop(0, nblk, blk_body, 0)

            @pl.when(nblk > 0)
            def _():
                scatter_wait((nblk - 1) % 2)
            plsc.subcore_barrier()

            pltpu.sync_copy(acc.at[pl.ds(s * ROWS_PT, ROWS_PT)],
                            out_hbm.at[pl.ds(lo + s * ROWS_PT, ROWS_PT)])
            plsc.subcore_barrier()

    cp = pltpu.CompilerParams()
    if "needs_layout_passes" in pltpu.CompilerParams.__dataclass_fields__:
        cp = dataclasses.replace(cp, needs_layout_passes=False)
    kern = pl.kernel(
        body,
        out_type=jax.ShapeDtypeStruct((N_PAD, D), jnp.float32),
        mesh=mesh,
        compiler_params=cp,
        scratch_types=[
            pltpu.VMEM((SCAN,), jnp.int32),
            pltpu.VMEM((SCAN,), jnp.int32),
            pltpu.VMEM((SCAN,), jnp.float32),
            pltpu.VMEM((CALLOC,), jnp.int32),
            pltpu.VMEM((CALLOC,), jnp.int32),
            pltpu.VMEM((CALLOC,), jnp.float32),
            pltpu.VMEM((2, BLK), jnp.int32),
            pltpu.VMEM((2, BLK), jnp.int32),
            pltpu.VMEM((2 * BLK, D), jnp.float32),
            pltpu.VMEM_SHARED((RANGE, D), jnp.float32),
            pltpu.SemaphoreType.DMA,
            pltpu.SemaphoreType.DMA,
        ],
    )
    return kern


_spmm128 = _make_spmm(128, 6400, 4, 8192)


def kernel(x, edge_index, edge_weight, W1, b1, W2, b2, W3, b3,
           A1, ab1, A2, ab2, A3, ab3, A4, ab4):
    dst = jnp.concatenate([edge_index[0],
                           jnp.zeros((E_PAD - E,), jnp.int32)])
    src = jnp.concatenate([edge_index[1],
                           jnp.zeros((E_PAD - E,), jnp.int32)])
    w = jnp.concatenate([edge_weight, jnp.zeros((E_PAD - E,), jnp.float32)])
    x_pad = jnp.zeros((N_PAD, 16), jnp.float32).at[:N, :DIN].set(x)
    z128 = jnp.zeros((6400, 128), jnp.float32)
    W1p = jnp.zeros((16, H), jnp.float32).at[:DIN].set(W1)

    s1 = _row_blocked(_dense0_body, H, x_pad, W1p)
    a1 = _spmm128(s1, dst, src, w, z128)
    s2 = _row_blocked(_dense2_body, H, a1, b1.reshape(1, H), W2)
    a2 = _spmm128(s2, dst, src, w, z128)
    s3 = _row_blocked(_dense2_body, H, a2, b2.reshape(1, H), W3)
    a3 = _spmm128(s3, dst, src, w, z128)
    scores = _row_blocked(_head_body, DOUT, a3,
                          b3.reshape(1, H), A1, ab1.reshape(1, H),
                          A2, ab2.reshape(1, H), A3, ab3.reshape(1, H),
                          A4, ab4.reshape(1, DOUT))
    return scores[:N]
